# baseline (speedup 1.0000x reference)
import jax
import jax.numpy as jnp
from jax.experimental import pallas as pl


def _blend_body(a_ref, x_ref, y_ref, e_ref, o_ref):
    a = a_ref[...]
    m = x_ref[...] * y_ref[...]
    o_ref[...] = a + m * (e_ref[...] - a)


def kernel(arr, element, x_index, y_index):
    M, N, D = arr.shape
    W = N * D
    a2 = arr.reshape(M, W)
    yb = jnp.repeat(y_index, D).reshape(1, W)
    eb = jnp.tile(element, N).reshape(1, W)
    x2 = x_index.reshape(M, 1)

    BM = min(256, M)
    BN = min(4096, W)
    grid = (M // BM, W // BN)

    out = pl.pallas_call(
        _blend_body,
        grid=grid,
        in_specs=[
            pl.BlockSpec((BM, BN), lambda i, j: (i, j)),
            pl.BlockSpec((BM, 1), lambda i, j: (i, 0)),
            pl.BlockSpec((1, BN), lambda i, j: (0, j)),
            pl.BlockSpec((1, BN), lambda i, j: (0, j)),
        ],
        out_specs=pl.BlockSpec((BM, BN), lambda i, j: (i, j)),
        out_shape=jax.ShapeDtypeStruct((M, W), jnp.float32),
    )(a2, x2, yb, eb)
    return out.reshape(M, N, D)
